# baseline (speedup 1.0000x reference)
import jax
import jax.numpy as jnp
from jax.experimental import pallas as pl
from jax.experimental.pallas import tpu as pltpu

NU = 10000
NI = 10000
E = 160000
D = 256
H = 4
DH = 64
DEA = 16
HID = 128

NB = 1000
NBLK = NU // NB


def _p1_body(a0, e0, d0, a1, e1, d1, W0, We0, W1r, We1r, W1, b1, w2r,
             z0o, z1o, s0o, s1o, acc):
    i = pl.program_id(0)

    @pl.when(i == 0)
    def _():
        acc[0] = 0.0
        acc[1] = 0.0

    rd0 = 1.0 / jnp.maximum(d0[...], 1.0)
    rd1 = 1.0 / jnp.maximum(d1[...], 1.0)
    z0 = (jnp.dot(a0[...], W0[...], preferred_element_type=jnp.float32)
          + jnp.dot(e0[...], We0[...], preferred_element_type=jnp.float32)) * rd0
    z1 = (jnp.dot(a1[...], W1r[...], preferred_element_type=jnp.float32)
          + jnp.dot(e1[...], We1r[...], preferred_element_type=jnp.float32)) * rd1
    h0 = jnp.tanh(jnp.dot(z0, W1[...], preferred_element_type=jnp.float32) + b1[...])
    h1 = jnp.tanh(jnp.dot(z1, W1[...], preferred_element_type=jnp.float32) + b1[...])
    acc[0] += jnp.sum(h0 * w2r[...])
    acc[1] += jnp.sum(h1 * w2r[...])
    z0o[...] = z0
    z1o[...] = z1

    @pl.when(i == NBLK - 1)
    def _():
        s0o[...] = jnp.full((1, 128), acc[0], jnp.float32)
        s1o[...] = jnp.full((1, 128), acc[1], jnp.float32)


def _p2_body(z0, z1, s0, s1, o):
    t0 = s0[0, 0] * (1.0 / NU)
    t1 = s1[0, 0] * (1.0 / NU)
    m = jnp.maximum(t0, t1)
    e0 = jnp.exp(t0 - m)
    e1 = jnp.exp(t1 - m)
    b0 = e0 / (e0 + e1)
    b1 = e1 / (e0 + e1)
    o[...] = b0 * z0[...] + b1 * z1[...]


def _dense_pair(a0, e0, d0, a1, e1, d1, W0, We0, W1r, We1r, W1, b1, w2):
    row = lambda i: (i, 0)
    const = lambda i: (0, 0)
    z0, z1, s0, s1 = pl.pallas_call(
        _p1_body,
        grid=(NBLK,),
        in_specs=[
            pl.BlockSpec((NB, D), row),
            pl.BlockSpec((NB, DEA), row),
            pl.BlockSpec((NB, 1), row),
            pl.BlockSpec((NB, D), row),
            pl.BlockSpec((NB, DEA), row),
            pl.BlockSpec((NB, 1), row),
            pl.BlockSpec((D, D), const),
            pl.BlockSpec((DEA, D), const),
            pl.BlockSpec((D, D), const),
            pl.BlockSpec((DEA, D), const),
            pl.BlockSpec((D, HID), const),
            pl.BlockSpec((1, HID), const),
            pl.BlockSpec((1, HID), const),
        ],
        out_specs=[
            pl.BlockSpec((NB, D), row),
            pl.BlockSpec((NB, D), row),
            pl.BlockSpec((1, 128), const),
            pl.BlockSpec((1, 128), const),
        ],
        out_shape=[
            jax.ShapeDtypeStruct((NU, D), jnp.float32),
            jax.ShapeDtypeStruct((NU, D), jnp.float32),
            jax.ShapeDtypeStruct((1, 128), jnp.float32),
            jax.ShapeDtypeStruct((1, 128), jnp.float32),
        ],
        scratch_shapes=[pltpu.SMEM((2,), jnp.float32)],
    )(a0, e0, d0.reshape(NU, 1), a1, e1, d1.reshape(NU, 1),
      W0, We0, W1r, We1r, W1, b1.reshape(1, HID), w2.reshape(1, HID))

    out = pl.pallas_call(
        _p2_body,
        grid=(NBLK,),
        in_specs=[
            pl.BlockSpec((NB, D), row),
            pl.BlockSpec((NB, D), row),
            pl.BlockSpec((1, 128), const),
            pl.BlockSpec((1, 128), const),
        ],
        out_specs=pl.BlockSpec((NB, D), row),
        out_shape=jax.ShapeDtypeStruct((NU, D), jnp.float32),
    )(z0, z1, s0, s1)
    return out.reshape(NU, H, DH)


def _seg(x_src, ei, ea, n_dst):
    src = ei[0]
    dst = ei[1]
    aggx = jax.ops.segment_sum(jnp.take(x_src, src, axis=0), dst, num_segments=n_dst)
    agge = jax.ops.segment_sum(ea, dst, num_segments=n_dst)
    deg = jax.ops.segment_sum(jnp.ones((ei.shape[1],), jnp.float32), dst,
                              num_segments=n_dst)
    return aggx, agge, deg


def kernel(x_user, x_item, ei_follows, ei_boughtby, ei_buys, ei_similar,
           ea_follows, ea_boughtby, ea_buys, ea_similar,
           W_follows, We_follows, W_boughtby, We_boughtby,
           W_buys, We_buys, W_similar, We_similar,
           W1_u, b1_u, w2_u, W1_i, b1_i, w2_i):
    af, ef, df = _seg(x_user, ei_follows, ea_follows, NU)
    ab, eb, db = _seg(x_item, ei_boughtby, ea_boughtby, NU)
    au, eu, du = _seg(x_user, ei_buys, ea_buys, NI)
    asim, es, ds = _seg(x_item, ei_similar, ea_similar, NI)

    out_user = _dense_pair(af, ef, df, ab, eb, db,
                           W_follows, We_follows, W_boughtby, We_boughtby,
                           W1_u, b1_u, w2_u)
    out_item = _dense_pair(au, eu, du, asim, es, ds,
                           W_buys, We_buys, W_similar, We_similar,
                           W1_i, b1_i, w2_i)
    return (out_user, out_item)
